# baseline (speedup 1.0000x reference)
import functools
import jax
import jax.numpy as jnp
from jax import lax
from jax.experimental import pallas as pl
from jax.experimental.pallas import tpu as pltpu
from jax.experimental.pallas import tpu_sc as plsc

D = 64
NW = 32
P = 200
NBUF = 2


def _emb_kernel(BR):
    RB = BR // NW
    SUBS = (128, 72)

    mesh = plsc.VectorSubcoreMesh(core_axis_name="c", subcore_axis_name="s")

    @functools.partial(
        pl.kernel,
        mesh=mesh,
        compiler_params=pltpu.CompilerParams(use_tc_tiling_on_sc=False),
        out_type=jax.ShapeDtypeStruct((BR, P, D), jnp.float32),
        scratch_types=[
            pltpu.VMEM((NBUF, P), jnp.int32),
            pltpu.VMEM((NBUF, P, D), jnp.float32),
            pltpu.SemaphoreType.DMA((NBUF,)),
            pltpu.SemaphoreType.DMA((NBUF,)),
            pltpu.SemaphoreType.DMA((NBUF,)),
        ],
    )
    def k(x_hbm, lut_hbm, out_hbm, idx_v, rows_v, sem_i, sem_g, sem_o):
        wid = lax.axis_index("s") * 2 + lax.axis_index("c")
        base = wid * RB

        def idx_copy(g, b):
            return pltpu.make_async_copy(
                x_hbm.at[pl.ds((base + g) * P, P)], idx_v.at[b], sem_i.at[b]
            )

        def gathers(b):
            cs = []
            off = 0
            for w in SUBS:
                cs.append(pltpu.make_async_copy(
                    lut_hbm.at[idx_v.at[b, pl.ds(off, w)]],
                    rows_v.at[b, pl.ds(off, w)],
                    sem_g.at[b],
                ))
                off += w
            return cs

        def out_copy(g, b):
            return pltpu.make_async_copy(
                rows_v.at[b], out_hbm.at[base + g], sem_o.at[b]
            )

        def compute(b):
            def grp_body(q, c2):
                r0 = q * 16
                xv = idx_v[b, pl.ds(r0, 16)]
                scv = jnp.where(xv == 0, jnp.float32(0.0), jnp.float32(8.0))
                for i in range(16):
                    sc = scv[i]
                    r = r0 + i
                    for j in range(D // 16):
                        sl = pl.ds(j * 16, 16)
                        rows_v[b, r, sl] = rows_v[b, r, sl] * sc
                return c2

            lax.fori_loop(0, P // 16, grp_body, 0)
            xv = idx_v[b, pl.ds(P - 16, 16)]
            scv = jnp.where(xv == 0, jnp.float32(0.0), jnp.float32(8.0))
            for i in range(8, 16):
                sc = scv[i]
                r = P - 16 + i
                for j in range(D // 16):
                    sl = pl.ds(j * 16, 16)
                    rows_v[b, r, sl] = rows_v[b, r, sl] * sc

        for b in range(NBUF):
            idx_copy(b, b).start()
        idx_copy(0, 0).wait()
        for c in gathers(0):
            c.start()

        def outer(o, carry):
            for b in range(NBUF):
                g = o * NBUF + b
                nb = (b + 1) % NBUF
                for c in gathers(b):
                    c.wait()
                @pl.when(g + 1 < RB)
                def _():
                    idx_copy(g + 1, nb).wait()

                    @pl.when(g + 1 >= NBUF)
                    def _():
                        out_copy(g + 1 - NBUF, nb).wait()

                    for c in gathers(nb):
                        c.start()

                compute(b)
                out_copy(g, b).start()

                @pl.when(g + NBUF < RB)
                def _():
                    idx_copy(g + NBUF, b).start()

            return carry

        lax.fori_loop(0, RB // NBUF, outer, 0)

        for b in range(NBUF):
            out_copy(RB - NBUF + b, b).wait()

    return k


def kernel(x, lut):
    BR = x.shape[0]
    xf = x.reshape(BR * P)
    return _emb_kernel(BR)(xf, lut)


# speedup vs baseline: 1.1223x; 1.1223x over previous
import functools
import jax
import jax.numpy as jnp
from jax import lax
from jax.experimental import pallas as pl
from jax.experimental.pallas import tpu as pltpu
from jax.experimental.pallas import tpu_sc as plsc

D = 64
NW = 32
G = 512
NSUB = G // 128
NBUF = 2


def _emb_kernel(B):
    R = B // NW
    N = R // G
    assert N % NBUF == 0

    mesh = plsc.VectorSubcoreMesh(core_axis_name="c", subcore_axis_name="s")

    @functools.partial(
        pl.kernel,
        mesh=mesh,
        compiler_params=pltpu.CompilerParams(use_tc_tiling_on_sc=False),
        out_type=jax.ShapeDtypeStruct((B, D), jnp.float32),
        scratch_types=[
            pltpu.VMEM((NBUF, NSUB, 128), jnp.int32),
            pltpu.VMEM((NBUF, G, D), jnp.float32),
            pltpu.SemaphoreType.DMA((NBUF,)),
            pltpu.SemaphoreType.DMA((NBUF,)),
            pltpu.SemaphoreType.DMA((NBUF,)),
        ],
    )
    def k(x_hbm, lut_hbm, out_hbm, idx_v, rows_v, sem_i, sem_g, sem_o):
        wid = lax.axis_index("s") * 2 + lax.axis_index("c")
        cbase = wid * N * NSUB

        def idx_copy(g, b):
            return pltpu.make_async_copy(
                x_hbm.at[pl.ds(cbase + g * NSUB, NSUB)], idx_v.at[b], sem_i.at[b]
            )

        def gathers(g, b):
            return [
                pltpu.make_async_copy(
                    lut_hbm.at[idx_v.at[b, j]],
                    rows_v.at[b, pl.ds(j * 128, 128)],
                    sem_g.at[b],
                )
                for j in range(NSUB)
            ]

        def out_copy(g, b):
            return pltpu.make_async_copy(
                rows_v.at[b], out_hbm.at[pl.ds((cbase + g * NSUB) * 128, G)],
                sem_o.at[b],
            )

        def compute(b):
            def grp_body(q, c2):
                r0 = q * 16
                jq = r0 // 128
                kq = r0 - jq * 128
                xv = idx_v[b, jq, pl.ds(kq, 16)]
                scv = jnp.where(xv == 0, jnp.float32(0.0), jnp.float32(8.0))
                for i in range(16):
                    sc = scv[i]
                    r = r0 + i
                    for j in range(D // 16):
                        sl = pl.ds(j * 16, 16)
                        rows_v[b, r, sl] = rows_v[b, r, sl] * sc
                return c2

            lax.fori_loop(0, G // 16, grp_body, 0)

        for b in range(NBUF):
            idx_copy(b, b).start()
        idx_copy(0, 0).wait()
        for c in gathers(0, 0):
            c.start()

        def outer(o, carry):
            for b in range(NBUF):
                g = o * NBUF + b
                nb = (b + 1) % NBUF
                for c in gathers(g, b):
                    c.wait()
                @pl.when(g + 1 < N)
                def _():
                    idx_copy(g + 1, nb).wait()

                    @pl.when(g + 1 >= NBUF)
                    def _():
                        out_copy(g + 1 - NBUF, nb).wait()

                    for c in gathers(g + 1, nb):
                        c.start()

                compute(b)
                out_copy(g, b).start()

                @pl.when(g + NBUF < N)
                def _():
                    idx_copy(g + NBUF, b).start()

            return carry

        lax.fori_loop(0, N // NBUF, outer, 0)

        for b in range(NBUF):
            g = N - NBUF + b
            out_copy(g, b).wait()

    return k


def kernel(x, lut):
    B = x.shape[0] * x.shape[1]
    xr = x.reshape(B // 128, 128)
    out = _emb_kernel(B)(xr, lut)
    return out.reshape(x.shape[0], x.shape[1], D)
